# Initial kernel scaffold; baseline (speedup 1.0000x reference)
import jax
import jax.numpy as jnp
from jax.experimental import pallas as pl


def kernel(query_clip, query_sbert, clip_features, sbert_features, k):
    raise NotImplementedError("write your pallas kernel here")


# speedup vs baseline: 1.3023x; 1.3023x over previous
import functools

import jax
import jax.numpy as jnp
from jax.experimental import pallas as pl
from jax.experimental.pallas import tpu as pltpu

TOPK = 10
KT = 512


def _topk_body(qc_ref, qs_ref, cf_ref, sf_ref, ov_ref, oi_ref,
               qcn, qsn, rv, ri, *, n_tiles, k_total, q, d):
    t = pl.program_id(0)

    @pl.when(t == 0)
    def _init():
        qc = qc_ref[...]
        qcn[...] = qc / (jnp.sqrt(jnp.sum(qc * qc, axis=1, keepdims=True)) + 1e-8)
        qs = qs_ref[...]
        qsn[...] = qs / (jnp.sqrt(jnp.sum(qs * qs, axis=1, keepdims=True)) + 1e-8)
        rv[...] = jnp.full((q, TOPK), -jnp.inf, jnp.float32)
        ri[...] = jnp.zeros((q, TOPK), jnp.int32)

    cf = cf_ref[...]
    cfn = cf / (jnp.sqrt(jnp.sum(cf * cf, axis=1, keepdims=True)) + 1e-8)
    sf = sf_ref[...]
    sfn = sf / (jnp.sqrt(jnp.sum(sf * sf, axis=1, keepdims=True)) + 1e-8)

    dn = (((1,), (1,)), ((), ()))
    s = (100.0 * jax.lax.dot_general(qcn[...], cfn, dn)
         + jax.lax.dot_general(qsn[...], sfn, dn))

    lane = jax.lax.broadcasted_iota(jnp.int32, (q, KT), 1)
    col0 = t * KT
    s = jnp.where(lane + col0 < k_total, s, -jnp.inf)

    big = jnp.int32(2**30)
    tvs, tis = [], []
    work = s
    for j in range(TOPK):
        m = jnp.max(work, axis=1, keepdims=True)
        a = jnp.min(jnp.where(work == m, lane, big), axis=1, keepdims=True)
        tvs.append(m)
        tis.append(a + col0)
        if j + 1 < TOPK:
            work = jnp.where(lane == a, -jnp.inf, work)
    tv = jnp.concatenate(tvs, axis=1)
    ti = jnp.concatenate(tis, axis=1)

    cv = jnp.concatenate([rv[...], tv], axis=1)
    ci = jnp.concatenate([ri[...], ti], axis=1)
    nvs, nis = [], []
    for j in range(TOPK):
        m = jnp.max(cv, axis=1, keepdims=True)
        p = jnp.min(jnp.where(cv == m, ci, big), axis=1, keepdims=True)
        nvs.append(m)
        nis.append(p)
        if j + 1 < TOPK:
            cv = jnp.where(ci == p, -jnp.inf, cv)
    nv = jnp.concatenate(nvs, axis=1)
    ni = jnp.concatenate(nis, axis=1)
    rv[...] = nv
    ri[...] = ni

    @pl.when(t == n_tiles - 1)
    def _flush():
        ov_ref[...] = nv
        oi_ref[...] = ni


def kernel(query_clip, query_sbert, clip_features, sbert_features, k):
    q, d = query_clip.shape
    k_total = clip_features.shape[0]
    n_tiles = pl.cdiv(k_total, KT)

    body = functools.partial(_topk_body, n_tiles=n_tiles, k_total=k_total,
                             q=q, d=d)
    vals, idx = pl.pallas_call(
        body,
        grid=(n_tiles,),
        in_specs=[
            pl.BlockSpec((q, d), lambda t: (0, 0)),
            pl.BlockSpec((q, d), lambda t: (0, 0)),
            pl.BlockSpec((KT, d), lambda t: (t, 0)),
            pl.BlockSpec((KT, d), lambda t: (t, 0)),
        ],
        out_specs=[
            pl.BlockSpec((q, TOPK), lambda t: (0, 0)),
            pl.BlockSpec((q, TOPK), lambda t: (0, 0)),
        ],
        out_shape=[
            jax.ShapeDtypeStruct((q, TOPK), jnp.float32),
            jax.ShapeDtypeStruct((q, TOPK), jnp.int32),
        ],
        scratch_shapes=[
            pltpu.VMEM((q, d), jnp.float32),
            pltpu.VMEM((q, d), jnp.float32),
            pltpu.VMEM((q, TOPK), jnp.float32),
            pltpu.VMEM((q, TOPK), jnp.int32),
        ],
        compiler_params=pltpu.CompilerParams(
            dimension_semantics=("arbitrary",),
        ),
    )(query_clip, query_sbert, clip_features, sbert_features)
    return vals, idx


# speedup vs baseline: 1.7638x; 1.3544x over previous
import functools

import jax
import jax.numpy as jnp
from jax.experimental import pallas as pl
from jax.experimental.pallas import tpu as pltpu

TOPK = 10
KT = 512
WC = 128
BIG = 2**30


def _topk_body(qc_ref, qs_ref, cf_ref, sf_ref, ov_ref, oi_ref,
               qcn, qsn, rv, ri, *, n_tiles, k_total, q, d):
    t = pl.program_id(0)

    @pl.when(t == 0)
    def _init():
        qc = qc_ref[...]
        qcn[...] = qc / (jnp.sqrt(jnp.sum(qc * qc, axis=1, keepdims=True)) + 1e-8)
        qs = qs_ref[...]
        qsn[...] = qs / (jnp.sqrt(jnp.sum(qs * qs, axis=1, keepdims=True)) + 1e-8)
        rv[...] = jnp.full((q, WC), -jnp.inf, jnp.float32)
        ri[...] = jnp.full((q, WC), BIG, jnp.int32)

    cf = cf_ref[...]
    cfn = cf / (jnp.sqrt(jnp.sum(cf * cf, axis=1, keepdims=True)) + 1e-8)
    sf = sf_ref[...]
    sfn = sf / (jnp.sqrt(jnp.sum(sf * sf, axis=1, keepdims=True)) + 1e-8)

    dn = (((1,), (1,)), ((), ()))
    s = (100.0 * jax.lax.dot_general(qcn[...], cfn, dn)
         + jax.lax.dot_general(qsn[...], sfn, dn))

    col0 = t * KT
    lane = jax.lax.broadcasted_iota(jnp.int32, (q, KT), 1)
    s = jnp.where(lane < k_total - col0, s, -jnp.inf)

    work = jnp.concatenate([s, rv[...]], axis=1)
    lidx = jnp.concatenate([lane, ri[...] - col0], axis=1)

    nvs, nis = [], []
    for j in range(TOPK):
        m = jnp.max(work, axis=1, keepdims=True)
        p = jnp.min(jnp.where(work == m, lidx, BIG), axis=1, keepdims=True)
        nvs.append(m)
        nis.append(p)
        if j + 1 < TOPK:
            work = jnp.where(lidx == p, -jnp.inf, work)
    nv = jnp.concatenate(nvs, axis=1)
    ni = jnp.concatenate(nis, axis=1) + col0
    rv[:, 0:TOPK] = nv
    ri[:, 0:TOPK] = ni

    @pl.when(t == n_tiles - 1)
    def _flush():
        ov_ref[...] = nv
        oi_ref[...] = ni


def kernel(query_clip, query_sbert, clip_features, sbert_features, k):
    q, d = query_clip.shape
    k_total = clip_features.shape[0]
    n_tiles = pl.cdiv(k_total, KT)

    body = functools.partial(_topk_body, n_tiles=n_tiles, k_total=k_total,
                             q=q, d=d)
    vals, idx = pl.pallas_call(
        body,
        grid=(n_tiles,),
        in_specs=[
            pl.BlockSpec((q, d), lambda t: (0, 0)),
            pl.BlockSpec((q, d), lambda t: (0, 0)),
            pl.BlockSpec((KT, d), lambda t: (t, 0)),
            pl.BlockSpec((KT, d), lambda t: (t, 0)),
        ],
        out_specs=[
            pl.BlockSpec((q, TOPK), lambda t: (0, 0)),
            pl.BlockSpec((q, TOPK), lambda t: (0, 0)),
        ],
        out_shape=[
            jax.ShapeDtypeStruct((q, TOPK), jnp.float32),
            jax.ShapeDtypeStruct((q, TOPK), jnp.int32),
        ],
        scratch_shapes=[
            pltpu.VMEM((q, d), jnp.float32),
            pltpu.VMEM((q, d), jnp.float32),
            pltpu.VMEM((q, WC), jnp.float32),
            pltpu.VMEM((q, WC), jnp.int32),
        ],
        compiler_params=pltpu.CompilerParams(
            dimension_semantics=("arbitrary",),
        ),
    )(query_clip, query_sbert, clip_features, sbert_features)
    return vals, idx


# speedup vs baseline: 2.3297x; 1.3209x over previous
import functools

import jax
import jax.numpy as jnp
from jax.experimental import pallas as pl
from jax.experimental.pallas import tpu as pltpu

TOPK = 10
KT = 512
WC = 128
FPAD = 2e6
FBIG = 4e6


def _topk_body(qc_ref, qs_ref, cf_ref, sf_ref, ov_ref, oi_ref,
               qcn, qsn, rv, ri, *, n_tiles, k_total, q, d):
    t = pl.program_id(0)

    @pl.when(t == 0)
    def _init():
        qc = qc_ref[...]
        qcn[...] = qc / (jnp.sqrt(jnp.sum(qc * qc, axis=1, keepdims=True)) + 1e-8)
        qs = qs_ref[...]
        qsn[...] = qs / (jnp.sqrt(jnp.sum(qs * qs, axis=1, keepdims=True)) + 1e-8)
        rv[...] = jnp.full((q, WC), -jnp.inf, jnp.float32)
        ri[...] = jnp.full((q, WC), FPAD, jnp.float32)

    cf = cf_ref[...]
    cfn = cf / (jnp.sqrt(jnp.sum(cf * cf, axis=1, keepdims=True)) + 1e-8)
    sf = sf_ref[...]
    sfn = sf / (jnp.sqrt(jnp.sum(sf * sf, axis=1, keepdims=True)) + 1e-8)

    dn = (((1,), (1,)), ((), ()))
    s = (100.0 * jax.lax.dot_general(qcn[...], cfn, dn)
         + jax.lax.dot_general(qsn[...], sfn, dn))

    col0 = t * KT
    fcol0 = col0.astype(jnp.float32)
    flane = jax.lax.broadcasted_iota(jnp.int32, (q, KT), 1).astype(jnp.float32)
    s = jnp.where(flane < (k_total - col0).astype(jnp.float32), s, -jnp.inf)

    work = jnp.concatenate([s, rv[...]], axis=1)
    flidx = jnp.concatenate([flane, ri[...] - fcol0], axis=1)

    nvs, nis = [], []
    for j in range(TOPK):
        m = jnp.max(work, axis=1, keepdims=True)
        p = jnp.min(jnp.where(work == m, flidx, FBIG), axis=1, keepdims=True)
        nvs.append(m)
        nis.append(p)
        if j + 1 < TOPK:
            work = jnp.where(flidx == p, -jnp.inf, work)
    nv = jnp.concatenate(nvs, axis=1)
    ni = jnp.concatenate(nis, axis=1) + fcol0
    rv[:, 0:TOPK] = nv
    ri[:, 0:TOPK] = ni

    @pl.when(t == n_tiles - 1)
    def _flush():
        ov_ref[...] = nv
        oi_ref[...] = ni.astype(jnp.int32)


def kernel(query_clip, query_sbert, clip_features, sbert_features, k):
    q, d = query_clip.shape
    k_total = clip_features.shape[0]
    n_tiles = pl.cdiv(k_total, KT)

    body = functools.partial(_topk_body, n_tiles=n_tiles, k_total=k_total,
                             q=q, d=d)
    vals, idx = pl.pallas_call(
        body,
        grid=(n_tiles,),
        in_specs=[
            pl.BlockSpec((q, d), lambda t: (0, 0)),
            pl.BlockSpec((q, d), lambda t: (0, 0)),
            pl.BlockSpec((KT, d), lambda t: (t, 0)),
            pl.BlockSpec((KT, d), lambda t: (t, 0)),
        ],
        out_specs=[
            pl.BlockSpec((q, TOPK), lambda t: (0, 0)),
            pl.BlockSpec((q, TOPK), lambda t: (0, 0)),
        ],
        out_shape=[
            jax.ShapeDtypeStruct((q, TOPK), jnp.float32),
            jax.ShapeDtypeStruct((q, TOPK), jnp.int32),
        ],
        scratch_shapes=[
            pltpu.VMEM((q, d), jnp.float32),
            pltpu.VMEM((q, d), jnp.float32),
            pltpu.VMEM((q, WC), jnp.float32),
            pltpu.VMEM((q, WC), jnp.float32),
        ],
        compiler_params=pltpu.CompilerParams(
            dimension_semantics=("arbitrary",),
        ),
    )(query_clip, query_sbert, clip_features, sbert_features)
    return vals, idx
